# Initial kernel scaffold; baseline (speedup 1.0000x reference)
import jax
import jax.numpy as jnp
from jax.experimental import pallas as pl


def kernel(x, w):
    raise NotImplementedError("write your pallas kernel here")


# speedup vs baseline: 1.1103x; 1.1103x over previous
import functools

import jax
import jax.numpy as jnp
from jax import lax
from jax.experimental import pallas as pl
from jax.experimental.pallas import tpu as pltpu
from jax.experimental.pallas import tpu_sc as plsc

EMBED_DIM = 32


@functools.partial(jax.jit, static_argnums=(2, 3, 4, 5))
def _gather_rows(idx, table, B, b_per_w, C, NC):
    mesh = plsc.VectorSubcoreMesh(core_axis_name="c", subcore_axis_name="s")

    @functools.partial(
        pl.kernel,
        mesh=mesh,
        out_type=jax.ShapeDtypeStruct((B, EMBED_DIM), jnp.float32),
        scratch_types=[
            pltpu.VMEM((C,), jnp.int32),
            pltpu.VMEM((C, EMBED_DIM), jnp.float32),
            pltpu.SemaphoreType.DMA,
        ],
        compiler_params=pltpu.CompilerParams(use_tc_tiling_on_sc=False),
    )
    def k(idx_hbm, table_hbm, out_hbm, idx_v, rows_v, sem):
        wid = lax.axis_index("s") * NC + lax.axis_index("c")
        base_w = wid * b_per_w

        def body(i, carry):
            base = pl.multiple_of(base_w + i * C, 8)
            pltpu.sync_copy(idx_hbm.at[pl.ds(base, C)], idx_v)
            pltpu.async_copy(table_hbm.at[idx_v], rows_v, sem).wait()
            pltpu.sync_copy(rows_v, out_hbm.at[pl.ds(base, C)])
            return carry

        lax.fori_loop(0, b_per_w // C, body, 0)

    return k(idx, table)


def kernel(x, w):
    B = x.shape[0] * x.shape[1]
    idx = x.reshape(B).astype(jnp.int32)
    info = plsc.get_sparse_core_info()
    NC, NS = info.num_cores, info.num_subcores
    b_per_w = B // (NC * NS)
    C = 3200
    out = _gather_rows(idx, w, B, b_per_w, C, NC)
    return out.reshape(x.shape[0], x.shape[1], EMBED_DIM)


# speedup vs baseline: 1.1132x; 1.0026x over previous
import functools

import jax
import jax.numpy as jnp
from jax import lax
from jax.experimental import pallas as pl
from jax.experimental.pallas import tpu as pltpu
from jax.experimental.pallas import tpu_sc as plsc

EMBED_DIM = 32


@functools.partial(jax.jit, static_argnums=(2, 3, 4, 5))
def _gather_rows(idx, table, B, b_per_w, C, NC):
    mesh = plsc.VectorSubcoreMesh(core_axis_name="c", subcore_axis_name="s")
    n = b_per_w // C

    @functools.partial(
        pl.kernel,
        mesh=mesh,
        out_type=jax.ShapeDtypeStruct((B, EMBED_DIM), jnp.float32),
        scratch_types=[
            pltpu.VMEM((C,), jnp.int32),
            pltpu.VMEM((C,), jnp.int32),
            pltpu.VMEM((C, EMBED_DIM), jnp.float32),
            pltpu.VMEM((C, EMBED_DIM), jnp.float32),
            pltpu.SemaphoreType.DMA,
            pltpu.SemaphoreType.DMA,
            pltpu.SemaphoreType.DMA,
            pltpu.SemaphoreType.DMA,
        ],
        compiler_params=pltpu.CompilerParams(use_tc_tiling_on_sc=False),
    )
    def k(idx_hbm, table_hbm, out_hbm, idx0, idx1, rows0, rows1, g0, g1, s0, s1):
        wid = lax.axis_index("s") * NC + lax.axis_index("c")
        base_w = wid * b_per_w

        def cbase(c):
            return pl.multiple_of(base_w + c * C, 8)

        pltpu.sync_copy(idx_hbm.at[pl.ds(cbase(0), C)], idx0)
        pltpu.async_copy(table_hbm.at[idx0], rows0, g0)
        pltpu.sync_copy(idx_hbm.at[pl.ds(cbase(1), C)], idx1)
        pltpu.async_copy(table_hbm.at[idx1], rows1, g1)

        def body(j, carry):
            a = 2 * j
            b = a + 1
            pltpu.make_async_copy(table_hbm.at[idx0], rows0, g0).wait()
            pltpu.async_copy(rows0, out_hbm.at[pl.ds(cbase(a), C)], s0)
            pltpu.make_async_copy(table_hbm.at[idx1], rows1, g1).wait()
            pltpu.async_copy(rows1, out_hbm.at[pl.ds(cbase(b), C)], s1)

            @pl.when(j < n // 2 - 1)
            def _():
                pltpu.sync_copy(idx_hbm.at[pl.ds(cbase(a + 2), C)], idx0)
                pltpu.make_async_copy(rows0, out_hbm.at[pl.ds(cbase(a), C)], s0).wait()
                pltpu.async_copy(table_hbm.at[idx0], rows0, g0)
                pltpu.sync_copy(idx_hbm.at[pl.ds(cbase(b + 2), C)], idx1)
                pltpu.make_async_copy(rows1, out_hbm.at[pl.ds(cbase(b), C)], s1).wait()
                pltpu.async_copy(table_hbm.at[idx1], rows1, g1)

            return carry

        lax.fori_loop(0, n // 2, body, 0)
        pltpu.make_async_copy(rows0, out_hbm.at[pl.ds(cbase(n - 2), C)], s0).wait()
        pltpu.make_async_copy(rows1, out_hbm.at[pl.ds(cbase(n - 1), C)], s1).wait()

    return k(idx, table)


def kernel(x, w):
    B = x.shape[0] * x.shape[1]
    idx = x.reshape(B).astype(jnp.int32)
    info = plsc.get_sparse_core_info()
    NC, NS = info.num_cores, info.num_subcores
    b_per_w = B // (NC * NS)
    C = 1600
    out = _gather_rows(idx, w, B, b_per_w, C, NC)
    return out.reshape(x.shape[0], x.shape[1], EMBED_DIM)
